# Initial kernel scaffold; baseline (speedup 1.0000x reference)
import jax
import jax.numpy as jnp
from jax.experimental import pallas as pl


def kernel(boxes, scores):
    raise NotImplementedError("write your pallas kernel here")


# speedup vs baseline: 18.1494x; 18.1494x over previous
import jax
import jax.numpy as jnp
from jax import lax
from jax.experimental import pallas as pl

_N = 20000
_LANES = 128
_ROWS = 160
_NP = _ROWS * _LANES
_PRE = 4096
_POST = 512
_IOU_THRESH = 0.7
_NEG = -1e30
_BIG_I = (1 << 30) - 1


def _nms_kernel(x1_ref, y1_ref, x2_ref, y2_ref, sc_ref, out_ref):
    x1 = x1_ref[...]
    y1 = y1_ref[...]
    x2 = x2_ref[...]
    y2 = y2_ref[...]
    sc = sc_ref[...]

    giota = (lax.broadcasted_iota(jnp.int32, (_ROWS, _LANES), 0) * _LANES
             + lax.broadcasted_iota(jnp.int32, (_ROWS, _LANES), 1))

    keys = lax.bitcast_convert_type(sc, jnp.int32)

    def bs_body(_, lohi):
        lo, hi = lohi
        mid = lo + (hi - lo) // 2
        cnt = jnp.sum((keys >= mid).astype(jnp.int32))
        ge = cnt >= _PRE
        return (jnp.where(ge, mid, lo), jnp.where(ge, hi, mid))

    lo, _ = lax.fori_loop(0, 31, bs_body, (jnp.int32(0), jnp.int32(1 << 30)))
    thresh_key = lo
    c_gt = jnp.sum((keys > thresh_key).astype(jnp.int32))
    r = _PRE - c_gt

    include = (keys > thresh_key).astype(jnp.int32)
    eqm = (keys == thresh_key).astype(jnp.int32)

    def tie_body(t, st):
        inc, eq = st
        idx = jnp.min(jnp.where(eq > 0, giota, _BIG_I))
        hit = ((giota == idx) & (t < r)).astype(jnp.int32)
        return (inc | hit, eq & (1 - hit))

    include, _ = lax.fori_loop(0, 16, tie_body, (include, eqm))

    work0 = jnp.where(include > 0, sc, _NEG)
    area = (x2 - x1) * (y2 - y1)
    lane = lax.broadcasted_iota(jnp.int32, (1, _LANES), 1)

    def step(i, st):
        work, fx1, fy1, fx2, fy2, fsc = st
        m = jnp.max(work)
        j = jnp.min(jnp.where(work == m, giota, _BIG_I))
        onehot = giota == j
        bx1 = jnp.max(jnp.where(onehot, x1, _NEG))
        by1 = jnp.max(jnp.where(onehot, y1, _NEG))
        bx2 = jnp.max(jnp.where(onehot, x2, _NEG))
        by2 = jnp.max(jnp.where(onehot, y2, _NEG))

        is_first = i == 0
        fx1 = jnp.where(is_first, bx1, fx1)
        fy1 = jnp.where(is_first, by1, fy1)
        fx2 = jnp.where(is_first, bx2, fx2)
        fy2 = jnp.where(is_first, by2, fy2)
        fsc = jnp.where(is_first, m, fsc)

        ix1 = jnp.maximum(bx1, x1)
        iy1 = jnp.maximum(by1, y1)
        ix2 = jnp.minimum(bx2, x2)
        iy2 = jnp.minimum(by2, y2)
        iw = jnp.maximum(ix2 - ix1, 0.0)
        ih = jnp.maximum(iy2 - iy1, 0.0)
        inter = iw * ih
        barea = (bx2 - bx1) * (by2 - by1)
        union = barea + area - inter
        iou = inter / jnp.maximum(union, 1e-8)
        suppress = (iou > _IOU_THRESH) | onehot
        work = jnp.where(suppress, _NEG, work)

        is_deg = m == _NEG
        ox1 = jnp.where(is_deg, fx1, bx1)
        oy1 = jnp.where(is_deg, fy1, by1)
        ox2 = jnp.where(is_deg, fx2, bx2)
        oy2 = jnp.where(is_deg, fy2, by2)
        osc = jnp.where(is_deg, fsc, m)

        row = jnp.where(lane == 0, ox1,
              jnp.where(lane == 1, oy1,
              jnp.where(lane == 2, ox2,
              jnp.where(lane == 3, oy2,
              jnp.where(lane == 4, osc, 0.0)))))
        out_ref[pl.ds(i, 1), :] = row
        return (work, fx1, fy1, fx2, fy2, fsc)

    zero = jnp.float32(0.0)
    lax.fori_loop(0, _POST, step, (work0, zero, zero, zero, zero, zero))


def kernel(boxes, scores):
    pad = _NP - _N
    x1 = jnp.pad(boxes[:, 0], (0, pad)).reshape(_ROWS, _LANES)
    y1 = jnp.pad(boxes[:, 1], (0, pad)).reshape(_ROWS, _LANES)
    x2 = jnp.pad(boxes[:, 2], (0, pad)).reshape(_ROWS, _LANES)
    y2 = jnp.pad(boxes[:, 3], (0, pad)).reshape(_ROWS, _LANES)
    sc = jnp.pad(scores, (0, pad), constant_values=-1.0).reshape(_ROWS, _LANES)
    out = pl.pallas_call(
        _nms_kernel,
        out_shape=jax.ShapeDtypeStruct((_POST, _LANES), jnp.float32),
    )(x1, y1, x2, y2, sc)
    return out[:, :5]


# speedup vs baseline: 18.7220x; 1.0315x over previous
import jax
import jax.numpy as jnp
from jax import lax
from jax.experimental import pallas as pl

_N = 20000
_LANES = 128
_ROWS = 160
_NP = _ROWS * _LANES
_PRE = 4096
_POST = 512
_IOU_THRESH = 0.7
_NEG = -1e30
_BIG_I = (1 << 30) - 1


def _nms_kernel(x1_ref, y1_ref, x2_ref, y2_ref, sc_ref, out_ref):
    x1 = x1_ref[...]
    y1 = y1_ref[...]
    x2 = x2_ref[...]
    y2 = y2_ref[...]
    sc = sc_ref[...]

    giota = (lax.broadcasted_iota(jnp.int32, (_ROWS, _LANES), 0) * _LANES
             + lax.broadcasted_iota(jnp.int32, (_ROWS, _LANES), 1))

    keys = lax.bitcast_convert_type(sc, jnp.int32)

    def bs_body(_, lohi):
        lo, hi = lohi
        mid = lo + (hi - lo) // 2
        cnt = jnp.sum((keys >= mid).astype(jnp.int32))
        ge = cnt >= _PRE
        return (jnp.where(ge, mid, lo), jnp.where(ge, hi, mid))

    lo, _ = lax.fori_loop(0, 31, bs_body, (jnp.int32(0), jnp.int32(1 << 30)))
    thresh_key = lo
    c_gt = jnp.sum((keys > thresh_key).astype(jnp.int32))
    r = _PRE - c_gt

    include = (keys > thresh_key).astype(jnp.int32)
    eqm = (keys == thresh_key).astype(jnp.int32)

    def tie_body(t, st):
        inc, eq = st
        idx = jnp.min(jnp.where(eq > 0, giota, _BIG_I))
        hit = ((giota == idx) & (t < r)).astype(jnp.int32)
        return (inc | hit, eq & (1 - hit))

    include, _ = lax.fori_loop(0, 16, tie_body, (include, eqm))

    work0 = jnp.where(include > 0, sc, _NEG)
    area = (x2 - x1) * (y2 - y1)
    lane = lax.broadcasted_iota(jnp.int32, (1, _LANES), 1)

    def step(i, st):
        work, fx1, fy1, fx2, fy2, fsc = st
        m = jnp.max(work)
        j = jnp.min(jnp.where(work == m, giota, _BIG_I))
        row = j // _LANES
        lj = j - row * _LANES
        lhot = lane == lj
        x1r = x1_ref[pl.ds(row, 1), :]
        y1r = y1_ref[pl.ds(row, 1), :]
        x2r = x2_ref[pl.ds(row, 1), :]
        y2r = y2_ref[pl.ds(row, 1), :]
        bx1 = jnp.max(jnp.where(lhot, x1r, _NEG))
        by1 = jnp.max(jnp.where(lhot, y1r, _NEG))
        bx2 = jnp.max(jnp.where(lhot, x2r, _NEG))
        by2 = jnp.max(jnp.where(lhot, y2r, _NEG))

        is_first = i == 0
        fx1 = jnp.where(is_first, bx1, fx1)
        fy1 = jnp.where(is_first, by1, fy1)
        fx2 = jnp.where(is_first, bx2, fx2)
        fy2 = jnp.where(is_first, by2, fy2)
        fsc = jnp.where(is_first, m, fsc)

        ix1 = jnp.maximum(bx1, x1)
        iy1 = jnp.maximum(by1, y1)
        ix2 = jnp.minimum(bx2, x2)
        iy2 = jnp.minimum(by2, y2)
        iw = jnp.maximum(ix2 - ix1, 0.0)
        ih = jnp.maximum(iy2 - iy1, 0.0)
        inter = iw * ih
        barea = (bx2 - bx1) * (by2 - by1)
        union = barea + area - inter
        iou = inter / jnp.maximum(union, 1e-8)
        suppress = (iou > _IOU_THRESH) | (giota == j)
        work = jnp.where(suppress, _NEG, work)

        is_deg = m == _NEG
        ox1 = jnp.where(is_deg, fx1, bx1)
        oy1 = jnp.where(is_deg, fy1, by1)
        ox2 = jnp.where(is_deg, fx2, bx2)
        oy2 = jnp.where(is_deg, fy2, by2)
        osc = jnp.where(is_deg, fsc, m)

        row = jnp.where(lane == 0, ox1,
              jnp.where(lane == 1, oy1,
              jnp.where(lane == 2, ox2,
              jnp.where(lane == 3, oy2,
              jnp.where(lane == 4, osc, 0.0)))))
        out_ref[pl.ds(i, 1), :] = row
        return (work, fx1, fy1, fx2, fy2, fsc)

    zero = jnp.float32(0.0)
    lax.fori_loop(0, _POST, step, (work0, zero, zero, zero, zero, zero))


def kernel(boxes, scores):
    pad = _NP - _N
    x1 = jnp.pad(boxes[:, 0], (0, pad)).reshape(_ROWS, _LANES)
    y1 = jnp.pad(boxes[:, 1], (0, pad)).reshape(_ROWS, _LANES)
    x2 = jnp.pad(boxes[:, 2], (0, pad)).reshape(_ROWS, _LANES)
    y2 = jnp.pad(boxes[:, 3], (0, pad)).reshape(_ROWS, _LANES)
    sc = jnp.pad(scores, (0, pad), constant_values=-1.0).reshape(_ROWS, _LANES)
    out = pl.pallas_call(
        _nms_kernel,
        out_shape=jax.ShapeDtypeStruct((_POST, _LANES), jnp.float32),
    )(x1, y1, x2, y2, sc)
    return out[:, :5]
